# baseline (speedup 1.0000x reference)
import jax
import jax.numpy as jnp
from jax import lax
from jax.experimental import pallas as pl
from jax.experimental.pallas import tpu as pltpu
from jax.experimental.pallas import tpu_sc as plsc

B, S, D = 32, 512, 768
HD = D // 2
T = B * S
NC, NS = 2, 16
NW = NC * NS
NCHUNK = 4
CT = T // NCHUNK
TOK_PER_W = CT // NW
F = 32
NSUB = TOK_PER_W // F
GRP = F * HD // 16
BT = 512
NBLK = CT // BT
EPS = 1e-12


def _pack_subchunk(rows_f, pack_b):
    def body(tok, _):
        for j in range(HD // 16):
            jb = j * 16
            a = rows_f[tok, pl.ds(jb, 16)]
            b = rows_f[tok, pl.ds(HD + jb, 16)]
            ai = lax.bitcast_convert_type(a, jnp.int32) + jnp.int32(0x8000)
            bi = lax.bitcast_convert_type(b, jnp.int32) + jnp.int32(0x8000)
            lo = lax.shift_right_logical(ai, jnp.int32(16))
            hi = bi & jnp.int32(-65536)
            pack_b[pl.ds(tok * HD + jb, 16)] = hi | lo
        return 0

    lax.fori_loop(0, F, body, 0)


def _sc_gather_body(table_hbm, ids_hbm, out_hbm, idx0, idx1, rows0, rows1,
                    pk0, pk1, sem_g0, sem_g1, sem_o):
    wid = lax.axis_index("s") * NC + lax.axis_index("c")
    base = wid * TOK_PER_W
    idxs = (idx0, idx1)
    rows = (rows0, rows1)
    pks = (pk0, pk1)
    sems = (sem_g0, sem_g1)

    def load_idx(f):
        pltpu.sync_copy(ids_hbm.at[pl.ds(base + f * F, F)], idxs[f % 2])

    def gather(f):
        return pltpu.make_async_copy(
            table_hbm.at[idxs[f % 2]], rows[f % 2], sems[f % 2]
        )

    def out_copy(f):
        return pltpu.make_async_copy(
            pks[f % 2], out_hbm.at[pl.ds((base + f * F) * HD, F * HD)], sem_o
        )

    load_idx(0)
    gather(0).start()
    for f in range(NSUB):
        if f + 1 < NSUB:
            load_idx(f + 1)
            gather(f + 1).start()
        gather(f).wait()
        if f >= 2:
            out_copy(f - 2).wait()
        _pack_subchunk(rows[f % 2], pks[f % 2])
        out_copy(f).start()
    out_copy(NSUB - 2).wait()
    out_copy(NSUB - 1).wait()


_sc_gather = pl.kernel(
    _sc_gather_body,
    out_type=jax.ShapeDtypeStruct((CT * HD,), jnp.int32),
    mesh=plsc.VectorSubcoreMesh(
        core_axis_name="c", subcore_axis_name="s", num_cores=NC, num_subcores=NS
    ),
    scratch_types=[
        pltpu.VMEM((F,), jnp.int32),
        pltpu.VMEM((F,), jnp.int32),
        pltpu.VMEM((F, D), jnp.float32),
        pltpu.VMEM((F, D), jnp.float32),
        pltpu.VMEM((F * HD,), jnp.int32),
        pltpu.VMEM((F * HD,), jnp.int32),
        pltpu.SemaphoreType.DMA,
        pltpu.SemaphoreType.DMA,
        pltpu.SemaphoreType.DMA,
    ],
)


def _make_tc_ln(chunk, aliased):
    def body(*refs):
        if aliased:
            refs = refs[1:]
        g_ref, tts_ref, eye_ref, pos_ref, wt_ref, gamma_ref, beta_ref, out_ref = refs
        t_row = tts_ref[0]
        t_col = lax.dot_general(
            eye_ref[...], t_row, (((1,), (1,)), ((), ())),
            preferred_element_type=jnp.float32,
        )
        w0 = wt_ref[0:1, :]
        diff = wt_ref[1:2, :] - w0
        packed = g_ref[...]
        lo = lax.bitcast_convert_type(
            lax.shift_left(packed, jnp.int32(16)), jnp.float32)
        hi = lax.bitcast_convert_type(
            packed & jnp.int32(-65536), jnp.float32)
        g = jnp.concatenate([lo, hi], axis=1)
        x = g + pos_ref[...] + w0 + t_col * diff
        s1 = jnp.sum(x, axis=-1, keepdims=True)
        s2 = jnp.sum(x * x, axis=-1, keepdims=True)
        mean = s1 * (1.0 / D)
        var = jnp.maximum(s2 * (1.0 / D) - mean * mean, 0.0)
        rstd = lax.rsqrt(var + EPS)
        out_ref[...] = (x - mean) * rstd * gamma_ref[...] + beta_ref[...]

    in_specs = [
        pl.BlockSpec((BT, HD), lambda i: (i, 0)),
        pl.BlockSpec((1, 1, BT), lambda i: (chunk * NBLK + i, 0, 0)),
        pl.BlockSpec((BT, BT), lambda i: (0, 0)),
        pl.BlockSpec((S, D), lambda i: (0, 0)),
        pl.BlockSpec((2, D), lambda i: (0, 0)),
        pl.BlockSpec((1, D), lambda i: (0, 0)),
        pl.BlockSpec((1, D), lambda i: (0, 0)),
    ]
    kwargs = {}
    if aliased:
        in_specs = [pl.BlockSpec(memory_space=pl.ANY)] + in_specs
        kwargs["input_output_aliases"] = {0: 0}
    return pl.pallas_call(
        body,
        grid=(NBLK,),
        in_specs=in_specs,
        out_specs=pl.BlockSpec((BT, D), lambda i: (chunk * NBLK + i, 0)),
        out_shape=jax.ShapeDtypeStruct((T, D), jnp.float32),
        **kwargs,
    )


def kernel(input_ids, token_type_ids, W_word, W_pos, W_type, gamma, beta):
    ids_flat = input_ids.reshape(T).astype(jnp.int32)
    tts = token_type_ids.reshape(B, 1, S).astype(jnp.float32)
    eye = jnp.eye(BT, dtype=jnp.float32)
    gamma2 = gamma.reshape(1, D)
    beta2 = beta.reshape(1, D)

    gathered = [
        _sc_gather(W_word, ids_flat[c * CT:(c + 1) * CT]).reshape(CT, HD)
        for c in range(NCHUNK)
    ]

    out = None
    for c in range(NCHUNK):
        args = (gathered[c], tts, eye, W_pos, W_type, gamma2, beta2)
        if c == 0:
            out = _make_tc_ln(c, aliased=False)(*args)
        else:
            out = _make_tc_ln(c, aliased=True)(out, *args)

    return out.reshape(B, S, D)


# speedup vs baseline: 1.2713x; 1.2713x over previous
import jax
import jax.numpy as jnp
from jax import lax
from jax.experimental import pallas as pl
from jax.experimental.pallas import tpu as pltpu
from jax.experimental.pallas import tpu_sc as plsc

B, S, D = 32, 512, 768
HD = D // 2
T = B * S
NC, NS = 2, 16
NW = NC * NS
NCHUNK = 4
CT = T // NCHUNK
TOK_PER_W = CT // NW
F = 32
NSUB = TOK_PER_W // F
GRP = F * HD // 16
BT = 512
NBLK = CT // BT
EPS = 1e-12


def _pack_subchunk(rows_f, pack_b):
    @plsc.parallel_loop(0, F, step=1, unroll=2)
    def _(tok):
        for j in range(HD // 16):
            jb = j * 16
            a = rows_f[tok, pl.ds(jb, 16)]
            b = rows_f[tok, pl.ds(HD + jb, 16)]
            ai = lax.bitcast_convert_type(a, jnp.int32) + jnp.int32(0x8000)
            bi = lax.bitcast_convert_type(b, jnp.int32) + jnp.int32(0x8000)
            lo = lax.shift_right_logical(ai, jnp.int32(16))
            hi = bi & jnp.int32(-65536)
            pack_b[pl.ds(tok * HD + jb, 16)] = hi | lo


def _sc_gather_body(table_hbm, ids_hbm, out_hbm, idx0, idx1, rows0, rows1,
                    pk0, pk1, sem_g0, sem_g1, sem_o):
    wid = lax.axis_index("s") * NC + lax.axis_index("c")
    base = wid * TOK_PER_W
    idxs = (idx0, idx1)
    rows = (rows0, rows1)
    pks = (pk0, pk1)
    sems = (sem_g0, sem_g1)

    def load_idx(f):
        pltpu.sync_copy(ids_hbm.at[pl.ds(base + f * F, F)], idxs[f % 2])

    def gather(f):
        return pltpu.make_async_copy(
            table_hbm.at[idxs[f % 2]], rows[f % 2], sems[f % 2]
        )

    def out_copy(f):
        return pltpu.make_async_copy(
            pks[f % 2], out_hbm.at[pl.ds((base + f * F) * HD, F * HD)], sem_o
        )

    load_idx(0)
    gather(0).start()
    for f in range(NSUB):
        if f + 1 < NSUB:
            load_idx(f + 1)
            gather(f + 1).start()
        gather(f).wait()
        if f >= 2:
            out_copy(f - 2).wait()
        _pack_subchunk(rows[f % 2], pks[f % 2])
        out_copy(f).start()
    out_copy(NSUB - 2).wait()
    out_copy(NSUB - 1).wait()


_sc_gather = pl.kernel(
    _sc_gather_body,
    out_type=jax.ShapeDtypeStruct((CT * HD,), jnp.int32),
    mesh=plsc.VectorSubcoreMesh(
        core_axis_name="c", subcore_axis_name="s", num_cores=NC, num_subcores=NS
    ),
    scratch_types=[
        pltpu.VMEM((F,), jnp.int32),
        pltpu.VMEM((F,), jnp.int32),
        pltpu.VMEM((F, D), jnp.float32),
        pltpu.VMEM((F, D), jnp.float32),
        pltpu.VMEM((F * HD,), jnp.int32),
        pltpu.VMEM((F * HD,), jnp.int32),
        pltpu.SemaphoreType.DMA,
        pltpu.SemaphoreType.DMA,
        pltpu.SemaphoreType.DMA,
    ],
)


def _make_tc_ln(chunk, aliased):
    def body(*refs):
        if aliased:
            refs = refs[1:]
        g_ref, tts_ref, eye_ref, pos_ref, wt_ref, gamma_ref, beta_ref, out_ref = refs
        t_row = tts_ref[0]
        t_col = lax.dot_general(
            eye_ref[...], t_row, (((1,), (1,)), ((), ())),
            preferred_element_type=jnp.float32,
        )
        w0 = wt_ref[0:1, :]
        diff = wt_ref[1:2, :] - w0
        packed = g_ref[...]
        lo = lax.bitcast_convert_type(
            lax.shift_left(packed, jnp.int32(16)), jnp.float32)
        hi = lax.bitcast_convert_type(
            packed & jnp.int32(-65536), jnp.float32)
        g = jnp.concatenate([lo, hi], axis=1)
        x = g + pos_ref[...] + w0 + t_col * diff
        s1 = jnp.sum(x, axis=-1, keepdims=True)
        s2 = jnp.sum(x * x, axis=-1, keepdims=True)
        mean = s1 * (1.0 / D)
        var = jnp.maximum(s2 * (1.0 / D) - mean * mean, 0.0)
        rstd = lax.rsqrt(var + EPS)
        out_ref[...] = (x - mean) * rstd * gamma_ref[...] + beta_ref[...]

    in_specs = [
        pl.BlockSpec((BT, HD), lambda i: (i, 0)),
        pl.BlockSpec((1, 1, BT), lambda i: (chunk * NBLK + i, 0, 0)),
        pl.BlockSpec((BT, BT), lambda i: (0, 0)),
        pl.BlockSpec((S, D), lambda i: (0, 0)),
        pl.BlockSpec((2, D), lambda i: (0, 0)),
        pl.BlockSpec((1, D), lambda i: (0, 0)),
        pl.BlockSpec((1, D), lambda i: (0, 0)),
    ]
    kwargs = {}
    if aliased:
        in_specs = [pl.BlockSpec(memory_space=pl.ANY)] + in_specs
        kwargs["input_output_aliases"] = {0: 0}
    return pl.pallas_call(
        body,
        grid=(NBLK,),
        in_specs=in_specs,
        out_specs=pl.BlockSpec((BT, D), lambda i: (chunk * NBLK + i, 0)),
        out_shape=jax.ShapeDtypeStruct((T, D), jnp.float32),
        **kwargs,
    )


def kernel(input_ids, token_type_ids, W_word, W_pos, W_type, gamma, beta):
    ids_flat = input_ids.reshape(T).astype(jnp.int32)
    tts = token_type_ids.reshape(B, 1, S).astype(jnp.float32)
    eye = jnp.eye(BT, dtype=jnp.float32)
    gamma2 = gamma.reshape(1, D)
    beta2 = beta.reshape(1, D)

    gathered = [
        _sc_gather(W_word, ids_flat[c * CT:(c + 1) * CT]).reshape(CT, HD)
        for c in range(NCHUNK)
    ]

    out = None
    for c in range(NCHUNK):
        args = (gathered[c], tts, eye, W_pos, W_type, gamma2, beta2)
        if c == 0:
            out = _make_tc_ln(c, aliased=False)(*args)
        else:
            out = _make_tc_ln(c, aliased=True)(out, *args)

    return out.reshape(B, S, D)


# speedup vs baseline: 1.5100x; 1.1878x over previous
import jax
import jax.numpy as jnp
from jax import lax
from jax.experimental import pallas as pl
from jax.experimental.pallas import tpu as pltpu
from jax.experimental.pallas import tpu_sc as plsc

B, S, D = 32, 512, 768
T = B * S
NC, NS = 2, 16
NW = NC * NS
NCHUNK = 4
CT = T // NCHUNK
TOK_PER_W = CT // NW
F = 64
NSUB = TOK_PER_W // F
BT = 512
NBLK = CT // BT
EPS = 1e-12


def _sc_gather_body(table_hbm, ids_hbm, out_hbm, idx_v, rows_v, sem):
    wid = lax.axis_index("s") * NC + lax.axis_index("c")
    base = wid * TOK_PER_W
    for f in range(NSUB):
        off = base + f * F
        pltpu.sync_copy(ids_hbm.at[pl.ds(off, F)], idx_v)
        pltpu.async_copy(table_hbm.at[idx_v], rows_v, sem).wait()
        pltpu.sync_copy(rows_v, out_hbm.at[pl.ds(off, F)])


_sc_gather = pl.kernel(
    _sc_gather_body,
    out_type=jax.ShapeDtypeStruct((CT, D), jnp.float32),
    mesh=plsc.VectorSubcoreMesh(
        core_axis_name="c", subcore_axis_name="s", num_cores=NC, num_subcores=NS
    ),
    scratch_types=[
        pltpu.VMEM((F,), jnp.int32),
        pltpu.VMEM((F, D), jnp.float32),
        pltpu.SemaphoreType.DMA,
    ],
)


def _make_tc_ln(chunk, aliased):
    def body(*refs):
        if aliased:
            refs = refs[1:]
        g_ref, tts_ref, eye_ref, pos_ref, wt_ref, gamma_ref, beta_ref, out_ref = refs
        t_row = tts_ref[0]
        t_col = lax.dot_general(
            eye_ref[...], t_row, (((1,), (1,)), ((), ())),
            preferred_element_type=jnp.float32,
        )
        w0 = wt_ref[0:1, :]
        diff = wt_ref[1:2, :] - w0
        x = g_ref[...] + pos_ref[...] + w0 + t_col * diff
        s1 = jnp.sum(x, axis=-1, keepdims=True)
        s2 = jnp.sum(x * x, axis=-1, keepdims=True)
        mean = s1 * (1.0 / D)
        var = jnp.maximum(s2 * (1.0 / D) - mean * mean, 0.0)
        rstd = lax.rsqrt(var + EPS)
        out_ref[...] = (x - mean) * rstd * gamma_ref[...] + beta_ref[...]

    in_specs = [
        pl.BlockSpec((BT, D), lambda i: (i, 0)),
        pl.BlockSpec((1, 1, BT), lambda i: (chunk * NBLK + i, 0, 0)),
        pl.BlockSpec((BT, BT), lambda i: (0, 0)),
        pl.BlockSpec((S, D), lambda i: (0, 0)),
        pl.BlockSpec((2, D), lambda i: (0, 0)),
        pl.BlockSpec((1, D), lambda i: (0, 0)),
        pl.BlockSpec((1, D), lambda i: (0, 0)),
    ]
    kwargs = {}
    if aliased:
        in_specs = [pl.BlockSpec(memory_space=pl.ANY)] + in_specs
        kwargs["input_output_aliases"] = {0: 0}
    return pl.pallas_call(
        body,
        grid=(NBLK,),
        in_specs=in_specs,
        out_specs=pl.BlockSpec((BT, D), lambda i: (chunk * NBLK + i, 0)),
        out_shape=jax.ShapeDtypeStruct((T, D), jnp.float32),
        **kwargs,
    )


def kernel(input_ids, token_type_ids, W_word, W_pos, W_type, gamma, beta):
    ids_flat = input_ids.reshape(T).astype(jnp.int32)
    tts = token_type_ids.reshape(B, 1, S).astype(jnp.float32)
    eye = jnp.eye(BT, dtype=jnp.float32)
    gamma2 = gamma.reshape(1, D)
    beta2 = beta.reshape(1, D)

    gathered = [
        _sc_gather(W_word, ids_flat[c * CT:(c + 1) * CT]) for c in range(NCHUNK)
    ]

    out = None
    for c in range(NCHUNK):
        args = (gathered[c], tts, eye, W_pos, W_type, gamma2, beta2)
        if c == 0:
            out = _make_tc_ln(c, aliased=False)(*args)
        else:
            out = _make_tc_ln(c, aliased=True)(out, *args)

    return out.reshape(B, S, D)
